# Initial kernel scaffold; baseline (speedup 1.0000x reference)
import jax
import jax.numpy as jnp
from jax.experimental import pallas as pl


def kernel(head_index, rel_index, tail_index, ent_emb, rel_emb):
    raise NotImplementedError("write your pallas kernel here")


# speedup vs baseline: 1.5097x; 1.5097x over previous
import functools

import jax
import jax.numpy as jnp
from jax import lax
from jax.experimental import pallas as pl
from jax.experimental.pallas import tpu as pltpu
from jax.experimental.pallas import tpu_sc as plsc

B = 16384
D = 128
LANES = 16
NGROUPS = D // LANES
CHUNK = 64
EPS = 1e-12


def _rsqrt_nr(x):
    i = lax.bitcast_convert_type(x, jnp.int32)
    i = jnp.int32(0x5F3759DF) - lax.shift_right_logical(i, 1)
    y = lax.bitcast_convert_type(i, jnp.float32)
    for _ in range(3):
        y = y * (1.5 - 0.5 * x * y * y)
    return y


def _sqrt_via_rsqrt(x):
    return x * _rsqrt_nr(x)


def _make_kernel():
    nc, ns = 2, 16
    nw = nc * ns
    bpw = B // nw
    nchunks = bpw // CHUNK
    mesh = plsc.VectorSubcoreMesh(
        core_axis_name="c", subcore_axis_name="s", num_cores=nc, num_subcores=ns
    )

    @functools.partial(
        pl.kernel,
        out_type=jax.ShapeDtypeStruct((B,), jnp.float32),
        mesh=mesh,
        compiler_params=pltpu.CompilerParams(needs_layout_passes=False),
        scratch_types=[
            pltpu.VMEM((bpw,), jnp.int32),
            pltpu.VMEM((bpw,), jnp.int32),
            pltpu.VMEM((bpw,), jnp.int32),
            pltpu.VMEM((2, CHUNK, D), jnp.float32),
            pltpu.VMEM((2, CHUNK, D), jnp.float32),
            pltpu.VMEM((2, CHUNK, D), jnp.float32),
            pltpu.VMEM((bpw,), jnp.float32),
            pltpu.SemaphoreType.DMA,
            pltpu.SemaphoreType.DMA,
        ],
    )
    def kern(head_hbm, rel_hbm, tail_hbm, ent_hbm, remb_hbm, out_hbm,
             hidx_v, ridx_v, tidx_v, hrows_v, rrows_v, trows_v, out_v,
             sem0, sem1):
        wid = lax.axis_index("s") * nc + lax.axis_index("c")
        base = wid * bpw
        pltpu.sync_copy(head_hbm.at[pl.ds(base, bpw)], hidx_v)
        pltpu.sync_copy(rel_hbm.at[pl.ds(base, bpw)], ridx_v)
        pltpu.sync_copy(tail_hbm.at[pl.ds(base, bpw)], tidx_v)

        sems = (sem0, sem1)

        def start(c):
            buf = c % 2
            sl = pl.ds(c * CHUNK, CHUNK)
            sem = sems[buf]
            return (
                pltpu.async_copy(ent_hbm.at[hidx_v.at[sl]], hrows_v.at[buf], sem),
                pltpu.async_copy(remb_hbm.at[ridx_v.at[sl]], rrows_v.at[buf], sem),
                pltpu.async_copy(ent_hbm.at[tidx_v.at[sl]], trows_v.at[buf], sem),
            )

        def compute(c):
            buf = c % 2
            lane0 = lax.iota(jnp.int32, LANES) == 0

            def body(i, carry):
                hs = [hrows_v[buf, i, pl.ds(g * LANES, LANES)] for g in range(NGROUPS)]
                ts = [trows_v[buf, i, pl.ds(g * LANES, LANES)] for g in range(NGROUPS)]
                rs = [rrows_v[buf, i, pl.ds(g * LANES, LANES)] for g in range(NGROUPS)]
                acc_h = hs[0] * hs[0]
                acc_t = ts[0] * ts[0]
                for g in range(1, NGROUPS):
                    acc_h = acc_h + hs[g] * hs[g]
                    acc_t = acc_t + ts[g] * ts[g]
                ssh = jnp.sum(acc_h)
                sst = jnp.sum(acc_t)
                inv_h = jnp.minimum(_rsqrt_nr(ssh), 1.0 / EPS)
                inv_t = jnp.minimum(_rsqrt_nr(sst), 1.0 / EPS)
                dv = hs[0] * inv_h + rs[0] - ts[0] * inv_t
                acc_c = dv * dv
                for g in range(1, NGROUPS):
                    dv = hs[g] * inv_h + rs[g] - ts[g] * inv_t
                    acc_c = acc_c + dv * dv
                ssc = jnp.sum(acc_c)
                res = jnp.zeros((LANES,), jnp.float32) + _sqrt_via_rsqrt(ssc)
                pos = jnp.zeros((LANES,), jnp.int32) + (c * CHUNK + i)
                plsc.store_scatter(out_v, [pos], res, mask=lane0)
                return carry

            lax.fori_loop(0, CHUNK, body, 0)

        descs = start(0)
        for c in range(nchunks):
            nxt = start(c + 1) if c + 1 < nchunks else ()
            for d in descs:
                d.wait()
            compute(c)
            descs = nxt

        pltpu.sync_copy(out_v, out_hbm.at[pl.ds(base, bpw)])

    return kern


@functools.cache
def _get_kernel():
    return _make_kernel()


def kernel(head_index, rel_index, tail_index, ent_emb, rel_emb):
    return _get_kernel()(
        head_index.astype(jnp.int32),
        rel_index.astype(jnp.int32),
        tail_index.astype(jnp.int32),
        ent_emb,
        rel_emb,
    )


# speedup vs baseline: 1.5987x; 1.0589x over previous
import functools

import jax
import jax.numpy as jnp
from jax import lax
from jax.experimental import pallas as pl
from jax.experimental.pallas import tpu as pltpu
from jax.experimental.pallas import tpu_sc as plsc

B = 16384
D = 128
LANES = 16
NGROUPS = D // LANES
CHUNK = 64
EPS = 1e-12


def _rsqrt_nr(x):
    i = lax.bitcast_convert_type(x, jnp.int32)
    i = jnp.int32(0x5F3759DF) - lax.shift_right_logical(i, 1)
    y = lax.bitcast_convert_type(i, jnp.float32)
    for _ in range(3):
        y = y * (1.5 - 0.5 * x * y * y)
    return y


def _sqrt_via_rsqrt(x):
    return x * _rsqrt_nr(x)


def _make_kernel():
    nc, ns = 2, 16
    nw = nc * ns
    bpw = B // nw
    nchunks = bpw // CHUNK
    mesh = plsc.VectorSubcoreMesh(
        core_axis_name="c", subcore_axis_name="s", num_cores=nc, num_subcores=ns
    )

    @functools.partial(
        pl.kernel,
        out_type=jax.ShapeDtypeStruct((B,), jnp.float32),
        mesh=mesh,
        compiler_params=pltpu.CompilerParams(needs_layout_passes=False),
        scratch_types=[
            pltpu.VMEM((bpw,), jnp.int32),
            pltpu.VMEM((bpw,), jnp.int32),
            pltpu.VMEM((bpw,), jnp.int32),
            pltpu.VMEM((2, CHUNK, D), jnp.float32),
            pltpu.VMEM((2, CHUNK, D), jnp.float32),
            pltpu.VMEM((2, CHUNK, D), jnp.float32),
            pltpu.VMEM((bpw,), jnp.float32),
            pltpu.SemaphoreType.DMA,
            pltpu.SemaphoreType.DMA,
        ],
    )
    def kern(head_hbm, rel_hbm, tail_hbm, ent_hbm, remb_hbm, out_hbm,
             hidx_v, ridx_v, tidx_v, hrows_v, rrows_v, trows_v, out_v,
             sem0, sem1):
        wid = lax.axis_index("s") * nc + lax.axis_index("c")
        base = wid * bpw
        pltpu.sync_copy(head_hbm.at[pl.ds(base, bpw)], hidx_v)
        pltpu.sync_copy(rel_hbm.at[pl.ds(base, bpw)], ridx_v)
        pltpu.sync_copy(tail_hbm.at[pl.ds(base, bpw)], tidx_v)

        sems = (sem0, sem1)
        tabs = (ent_hbm, remb_hbm, ent_hbm)
        idxs = (hidx_v, ridx_v, tidx_v)
        rows = (hrows_v, rrows_v, trows_v)

        def descs(c, buf):
            sl = pl.ds(c * CHUNK, CHUNK)
            return [
                pltpu.make_async_copy(
                    tabs[k].at[idxs[k].at[sl]], rows[k].at[buf], sems[buf]
                )
                for k in range(3)
            ]

        def start(c, buf):
            for d in descs(c, buf):
                d.start()

        def wait(c, buf):
            for d in descs(c, buf):
                d.wait()

        def compute(c, buf):
            lane0 = lax.iota(jnp.int32, LANES) == 0

            def body(i, carry):
                hs = [hrows_v[buf, i, pl.ds(g * LANES, LANES)] for g in range(NGROUPS)]
                ts = [trows_v[buf, i, pl.ds(g * LANES, LANES)] for g in range(NGROUPS)]
                rs = [rrows_v[buf, i, pl.ds(g * LANES, LANES)] for g in range(NGROUPS)]
                acc_h = hs[0] * hs[0]
                acc_t = ts[0] * ts[0]
                for g in range(1, NGROUPS):
                    acc_h = acc_h + hs[g] * hs[g]
                    acc_t = acc_t + ts[g] * ts[g]
                ssh = jnp.sum(acc_h)
                sst = jnp.sum(acc_t)
                inv_h = jnp.minimum(_rsqrt_nr(ssh), 1.0 / EPS)
                inv_t = jnp.minimum(_rsqrt_nr(sst), 1.0 / EPS)
                dv = hs[0] * inv_h + rs[0] - ts[0] * inv_t
                acc_c = dv * dv
                for g in range(1, NGROUPS):
                    dv = hs[g] * inv_h + rs[g] - ts[g] * inv_t
                    acc_c = acc_c + dv * dv
                ssc = jnp.sum(acc_c)
                res = jnp.zeros((LANES,), jnp.float32) + _sqrt_via_rsqrt(ssc)
                pos = jnp.zeros((LANES,), jnp.int32) + (c * CHUNK + i)
                plsc.store_scatter(out_v, [pos], res, mask=lane0)
                return carry

            lax.fori_loop(0, CHUNK, body, 0, unroll=4)

        start(0, 0)

        def pair(p, carry):
            c0 = 2 * p
            start(c0 + 1, 1)
            wait(c0, 0)
            compute(c0, 0)

            @pl.when(c0 + 2 < nchunks)
            def _():
                start(c0 + 2, 0)

            wait(c0 + 1, 1)
            compute(c0 + 1, 1)
            return carry

        lax.fori_loop(0, nchunks // 2, pair, 0)

        pltpu.sync_copy(out_v, out_hbm.at[pl.ds(base, bpw)])

    return kern


@functools.cache
def _get_kernel():
    return _make_kernel()


def kernel(head_index, rel_index, tail_index, ent_emb, rel_emb):
    return _get_kernel()(
        head_index.astype(jnp.int32),
        rel_index.astype(jnp.int32),
        tail_index.astype(jnp.int32),
        ent_emb,
        rel_emb,
    )
